# baseline (speedup 1.0000x reference)
import functools

import jax
import jax.numpy as jnp
from jax import lax
from jax.experimental import pallas as pl
from jax.experimental.pallas import tpu as pltpu
from jax.experimental.pallas import tpu_sc as plsc

_D_MODEL = 4096
_RANK = 64
_EXPERTS = 64
_K = 8
_BLOCK = 1024

_NC = 2
_NS = 16
_NW = _NC * _NS
_TOKENS = 32768
_TPW = _TOKENS // _NW


def _scores_kernel(x_ref, u_ref, v_ref, b_ref, p_ref):
    x = x_ref[...]
    xu = jnp.dot(x, u_ref[...], preferred_element_type=jnp.float32)
    s = jnp.dot(xu, v_ref[...], preferred_element_type=jnp.float32)
    s = s + b_ref[...]
    m = jnp.max(s, axis=-1, keepdims=True)
    e = jnp.exp(s - m)
    p_ref[...] = e / jnp.sum(e, axis=-1, keepdims=True)


_CHUNK = 256


def _sc_router(p_hbm, w_hbm, i_hbm, pbuf, ibuf):
    wid = lax.axis_index("s") * _NC + lax.axis_index("c")
    base = wid * _TPW

    iota = lax.iota(jnp.int32, 16)

    def body(t, carry):
        ps = [pbuf[t, pl.ds(16 * j, 16)] for j in range(4)]
        for j in range(4):
            pbuf[t, pl.ds(16 * j, 16)] = ps[j] + 1.0
        ibuf[pl.ds(t * _K, 16)] = iota
        return carry

    def chunk(c, carry):
        tok0 = base + c * _CHUNK
        pltpu.sync_copy(p_hbm.at[pl.ds(tok0, _CHUNK)], pbuf)
        lax.fori_loop(0, _CHUNK, body, jnp.int32(0))
        pltpu.sync_copy(pbuf, w_hbm.at[pl.ds(tok0, _CHUNK)])
        pltpu.sync_copy(ibuf.at[pl.ds(0, _CHUNK * _K)],
                        i_hbm.at[pl.ds(tok0 * _K, _CHUNK * _K)])
        return carry

    lax.fori_loop(0, _TPW // _CHUNK, chunk, jnp.int32(0))


_sc_router_call = functools.partial(
    pl.kernel,
    mesh=plsc.VectorSubcoreMesh(core_axis_name="c", subcore_axis_name="s"),
    out_type=[
        jax.ShapeDtypeStruct((_TOKENS, _EXPERTS), jnp.float32),
        jax.ShapeDtypeStruct((_TOKENS * _K,), jnp.int32),
    ],
    scratch_types=[
        pltpu.VMEM((_CHUNK, _EXPERTS), jnp.float32),
        pltpu.VMEM((_CHUNK * _K + 8,), jnp.int32),
    ],
)(_sc_router)


def kernel(prime_memory_output, U_route, V_route, routing_bias, top_k):
    tokens = prime_memory_output.shape[0]
    grid = (tokens // _BLOCK,)
    bias2d = routing_bias.reshape(1, _EXPERTS)
    probs = pl.pallas_call(
        _scores_kernel,
        grid=grid,
        in_specs=[
            pl.BlockSpec((_BLOCK, _D_MODEL), lambda i: (i, 0)),
            pl.BlockSpec((_D_MODEL, _RANK), lambda i: (0, 0)),
            pl.BlockSpec((_RANK, _EXPERTS), lambda i: (0, 0)),
            pl.BlockSpec((1, _EXPERTS), lambda i: (0, 0)),
        ],
        out_specs=pl.BlockSpec((_BLOCK, _EXPERTS), lambda i: (i, 0)),
        out_shape=jax.ShapeDtypeStruct((tokens, _EXPERTS), jnp.float32),
        compiler_params=pltpu.CompilerParams(
            dimension_semantics=("parallel",),
        ),
    )(prime_memory_output, U_route, V_route, bias2d)
    weights, indices_flat = _sc_router_call(probs)
    return weights, indices_flat.reshape(tokens, _K)


# speedup vs baseline: 1.1397x; 1.1397x over previous
import jax
import jax.numpy as jnp
from jax.experimental import pallas as pl
from jax.experimental.pallas import tpu as pltpu

_D_MODEL = 4096
_RANK = 64
_EXPERTS = 64
_K = 8
_BLOCK = 1024


def _router_block_kernel(x_ref, u_ref, v_ref, b_ref, w_ref, i_ref):
    x = x_ref[...]
    xu = jnp.dot(x, u_ref[...], preferred_element_type=jnp.float32)
    s = jnp.dot(xu, v_ref[...], preferred_element_type=jnp.float32)
    s = s + b_ref[...]
    m = jnp.max(s, axis=-1, keepdims=True)
    e = jnp.exp(s - m)
    p = e / jnp.sum(e, axis=-1, keepdims=True)

    lane = jax.lax.broadcasted_iota(jnp.int32, s.shape, 1).astype(jnp.float32)
    col = jax.lax.broadcasted_iota(jnp.int32, (s.shape[0], _K), 1).astype(jnp.float32)
    neg_inf = jnp.float32(-jnp.inf)
    work = s
    idx_out = jnp.zeros((s.shape[0], _K), jnp.float32)
    for j in range(_K):
        mx = jnp.max(work, axis=-1, keepdims=True)
        idx = jnp.min(jnp.where(work == mx, lane, float(_EXPERTS)),
                      axis=-1, keepdims=True)
        work = jnp.where(lane == idx, neg_inf, work)
        idx_out = jnp.where(col == j, idx, idx_out)

    w_ref[...] = jnp.where(work == neg_inf, p, 0.0)
    i_ref[...] = idx_out.astype(jnp.int32)


def kernel(prime_memory_output, U_route, V_route, routing_bias, top_k):
    tokens = prime_memory_output.shape[0]
    grid = (tokens // _BLOCK,)
    bias2d = routing_bias.reshape(1, _EXPERTS)
    weights, indices = pl.pallas_call(
        _router_block_kernel,
        grid=grid,
        in_specs=[
            pl.BlockSpec((_BLOCK, _D_MODEL), lambda i: (i, 0)),
            pl.BlockSpec((_D_MODEL, _RANK), lambda i: (0, 0)),
            pl.BlockSpec((_RANK, _EXPERTS), lambda i: (0, 0)),
            pl.BlockSpec((1, _EXPERTS), lambda i: (0, 0)),
        ],
        out_specs=[
            pl.BlockSpec((_BLOCK, _EXPERTS), lambda i: (i, 0)),
            pl.BlockSpec((_BLOCK, _K), lambda i: (i, 0)),
        ],
        out_shape=[
            jax.ShapeDtypeStruct((tokens, _EXPERTS), jnp.float32),
            jax.ShapeDtypeStruct((tokens, _K), jnp.int32),
        ],
        compiler_params=pltpu.CompilerParams(
            dimension_semantics=("parallel",),
        ),
    )(prime_memory_output, U_route, V_route, bias2d)
    return weights, indices


# speedup vs baseline: 1.2075x; 1.0595x over previous
import jax
import jax.numpy as jnp
from jax.experimental import pallas as pl
from jax.experimental.pallas import tpu as pltpu

_D_MODEL = 4096
_RANK = 64
_EXPERTS = 64
_K = 8
_BLOCK = 1024


def _router_block_kernel(x_ref, u_ref, v_ref, b_ref, w_ref, i_ref):
    x = x_ref[...]
    xu = jnp.dot(x, u_ref[...], preferred_element_type=jnp.float32)
    s = jnp.dot(xu, v_ref[...], preferred_element_type=jnp.float32)
    s = s + b_ref[...]
    m = jnp.max(s, axis=-1, keepdims=True)
    e = jnp.exp(s - m)
    p = e / jnp.sum(e, axis=-1, keepdims=True)

    lane = jax.lax.broadcasted_iota(jnp.int32, s.shape, 1)
    col = jax.lax.broadcasted_iota(jnp.int32, (s.shape[0], _K), 1)
    neg_inf = jnp.float32(-jnp.inf)
    work = s
    idx_out = jnp.zeros((s.shape[0], _K), jnp.int32)
    for j in range(_K):
        idx = jnp.argmax(work, axis=-1, keepdims=True)
        work = jnp.where(lane == idx, neg_inf, work)
        idx_out = jnp.where(col == j, idx, idx_out)

    w_ref[...] = jnp.where(work == neg_inf, p, 0.0)
    i_ref[...] = idx_out


def kernel(prime_memory_output, U_route, V_route, routing_bias, top_k):
    tokens = prime_memory_output.shape[0]
    grid = (tokens // _BLOCK,)
    bias2d = routing_bias.reshape(1, _EXPERTS)
    weights, indices = pl.pallas_call(
        _router_block_kernel,
        grid=grid,
        in_specs=[
            pl.BlockSpec((_BLOCK, _D_MODEL), lambda i: (i, 0)),
            pl.BlockSpec((_D_MODEL, _RANK), lambda i: (0, 0)),
            pl.BlockSpec((_RANK, _EXPERTS), lambda i: (0, 0)),
            pl.BlockSpec((1, _EXPERTS), lambda i: (0, 0)),
        ],
        out_specs=[
            pl.BlockSpec((_BLOCK, _EXPERTS), lambda i: (i, 0)),
            pl.BlockSpec((_BLOCK, _K), lambda i: (i, 0)),
        ],
        out_shape=[
            jax.ShapeDtypeStruct((tokens, _EXPERTS), jnp.float32),
            jax.ShapeDtypeStruct((tokens, _K), jnp.int32),
        ],
        compiler_params=pltpu.CompilerParams(
            dimension_semantics=("parallel",),
        ),
    )(prime_memory_output, U_route, V_route, bias2d)
    return weights, indices
